# speedup vs baseline: 2.4930x; 2.4930x over previous
import jax
import jax.numpy as jnp
from jax.experimental import pallas as pl

_TOP_K = 2
_TILE = 2048


def _router_kernel(x_ref, w_ref, idx_ref, gate_ref):
    i = pl.program_id(0)
    x = x_ref[...]
    w = w_ref[...]
    logits = jax.lax.dot_general(
        x, w, (((1,), (1,)), ((), ())),
        preferred_element_type=jnp.float32)

    m1 = jnp.max(logits, axis=1)
    i1 = jnp.argmax(logits, axis=1).astype(jnp.int32)
    col = jax.lax.broadcasted_iota(jnp.int32, logits.shape, 1)
    masked = jnp.where(col == i1[:, None], -jnp.inf, logits)
    m2 = jnp.max(masked, axis=1)
    i2 = jnp.argmax(masked, axis=1).astype(jnp.int32)

    t = jnp.exp(m2 - m1)
    g1 = 1.0 / (1.0 + t)
    g2 = t / (1.0 + t)

    base = i * _TILE
    idx_ref[pl.ds(base, _TILE), :] = jnp.stack([i1, i2], axis=1)
    gate_ref[pl.ds(base, _TILE), :] = jnp.stack([g1, g2], axis=1)


@jax.jit
def kernel(x, W):
    n, d = x.shape
    e = W.shape[0]
    grid = (n // _TILE,)
    idx, gates = pl.pallas_call(
        _router_kernel,
        grid=grid,
        in_specs=[
            pl.BlockSpec((_TILE, d), lambda i: (i, 0)),
            pl.BlockSpec((e, d), lambda i: (0, 0)),
        ],
        out_specs=[
            pl.BlockSpec((n, _TOP_K), lambda i: (0, 0)),
            pl.BlockSpec((n, _TOP_K), lambda i: (0, 0)),
        ],
        out_shape=[
            jax.ShapeDtypeStruct((n, _TOP_K), jnp.int32),
            jax.ShapeDtypeStruct((n, _TOP_K), jnp.float32),
        ],
    )(x, W)
    return idx, gates


# speedup vs baseline: 2.5560x; 1.0253x over previous
import jax
import jax.numpy as jnp
from jax.experimental import pallas as pl
from jax.experimental.pallas import tpu as pltpu

_TOP_K = 2
_TILE = 2048


def _router_kernel(x_ref, w_ref, idx_ref, gate_ref):
    x = x_ref[...]
    w = w_ref[...]
    logits = jax.lax.dot_general(
        x, w, (((1,), (1,)), ((), ())),
        preferred_element_type=jnp.float32)

    m1 = jnp.max(logits, axis=1)
    i1 = jnp.argmax(logits, axis=1).astype(jnp.int32)
    col = jax.lax.broadcasted_iota(jnp.int32, logits.shape, 1)
    masked = jnp.where(col == i1[:, None], -jnp.inf, logits)
    m2 = jnp.max(masked, axis=1)
    i2 = jnp.argmax(masked, axis=1).astype(jnp.int32)

    t = jnp.exp(m2 - m1)
    g1 = 1.0 / (1.0 + t)
    g2 = t / (1.0 + t)

    idx_ref[...] = jnp.stack([i1, i2], axis=1)
    gate_ref[...] = jnp.stack([g1, g2], axis=1)


@jax.jit
def kernel(x, W):
    n, d = x.shape
    e = W.shape[0]
    grid = (n // _TILE,)
    idx, gates = pl.pallas_call(
        _router_kernel,
        grid=grid,
        in_specs=[
            pl.BlockSpec((_TILE, d), lambda i: (i, 0)),
            pl.BlockSpec((e, d), lambda i: (0, 0)),
        ],
        out_specs=[
            pl.BlockSpec((_TILE, _TOP_K), lambda i: (i, 0)),
            pl.BlockSpec((_TILE, _TOP_K), lambda i: (i, 0)),
        ],
        out_shape=[
            jax.ShapeDtypeStruct((n, _TOP_K), jnp.int32),
            jax.ShapeDtypeStruct((n, _TOP_K), jnp.float32),
        ],
        compiler_params=pltpu.CompilerParams(
            dimension_semantics=("parallel",)),
    )(x, W)
    return idx, gates


# speedup vs baseline: 2.6913x; 1.0529x over previous
import jax
import jax.numpy as jnp
from jax.experimental import pallas as pl
from jax.experimental.pallas import tpu as pltpu

_TOP_K = 2
_TILE = 4096


def _router_kernel(x_ref, w_ref, idx_ref, gate_ref):
    x = x_ref[...]
    w = w_ref[...]
    logits = jax.lax.dot_general(
        x, w, (((1,), (1,)), ((), ())),
        preferred_element_type=jnp.float32)

    m1 = jnp.max(logits, axis=1)
    i1 = jnp.argmax(logits, axis=1).astype(jnp.int32)
    col = jax.lax.broadcasted_iota(jnp.int32, logits.shape, 1)
    masked = jnp.where(col == i1[:, None], -jnp.inf, logits)
    m2 = jnp.max(masked, axis=1)
    i2 = jnp.argmax(masked, axis=1).astype(jnp.int32)

    t = jnp.exp(m2 - m1)
    g1 = 1.0 / (1.0 + t)
    g2 = t / (1.0 + t)

    idx_ref[...] = jnp.stack([i1, i2], axis=1)
    gate_ref[...] = jnp.stack([g1, g2], axis=1)


@jax.jit
def kernel(x, W):
    n, d = x.shape
    e = W.shape[0]
    grid = (n // _TILE,)
    idx, gates = pl.pallas_call(
        _router_kernel,
        grid=grid,
        in_specs=[
            pl.BlockSpec((_TILE, d), lambda i: (i, 0)),
            pl.BlockSpec((e, d), lambda i: (0, 0)),
        ],
        out_specs=[
            pl.BlockSpec((_TILE, _TOP_K), lambda i: (i, 0)),
            pl.BlockSpec((_TILE, _TOP_K), lambda i: (i, 0)),
        ],
        out_shape=[
            jax.ShapeDtypeStruct((n, _TOP_K), jnp.int32),
            jax.ShapeDtypeStruct((n, _TOP_K), jnp.float32),
        ],
        compiler_params=pltpu.CompilerParams(
            dimension_semantics=("parallel",)),
    )(x, W)
    return idx, gates
